# Initial kernel scaffold; baseline (speedup 1.0000x reference)
import jax
import jax.numpy as jnp
from jax.experimental import pallas as pl


def kernel(feature, edge_index, W, b):
    raise NotImplementedError("write your pallas kernel here")


# speedup vs baseline: 5.4640x; 5.4640x over previous
import functools

import jax
import jax.numpy as jnp
from jax import lax
from jax.experimental import pallas as pl
from jax.experimental.pallas import tpu as pltpu
from jax.experimental.pallas import tpu_sc as plsc

N_NODES = 10000
N_EDGES = 320000
D = 128

NC = 2
NS = 16
NW = NC * NS
EPT = N_EDGES // NW
CH = 80
NCHUNK = EPT // CH
N_PAD = 10240
ROWS_PT = N_PAD // NS
ZROWS = 128
LANES = 16


def _sc_segment_sum(feature, src, dst):
    mesh = plsc.VectorSubcoreMesh(core_axis_name="c", subcore_axis_name="s")

    @functools.partial(
        pl.kernel,
        mesh=mesh,
        out_type=jax.ShapeDtypeStruct((NC, N_PAD, D), jnp.float32),
        scratch_types=[
            pltpu.VMEM((CH,), jnp.int32),
            pltpu.VMEM((CH,), jnp.int32),
            pltpu.VMEM((CH, D), jnp.float32),
            pltpu.VMEM((ZROWS, D), jnp.float32),
            pltpu.VMEM_SHARED((N_PAD, D), jnp.float32),
            pltpu.SemaphoreType.DMA,
        ],
    )
    def k(feat_hbm, src_hbm, dst_hbm, out_hbm,
          sidx_v, didx_v, rows_v, stage_v, acc_sh, sem):
        cid = lax.axis_index("c")
        sid = lax.axis_index("s")
        wid = sid * NC + cid

        zero = jnp.zeros((LANES,), jnp.float32)

        def zbody(i, carry):
            r = i // (D // LANES)
            col = (i % (D // LANES)) * LANES
            stage_v[r, pl.ds(col, LANES)] = zero
            return carry

        lax.fori_loop(0, ZROWS * (D // LANES), zbody, 0)

        row0 = sid * ROWS_PT

        def zcopy(j, carry):
            pltpu.sync_copy(stage_v, acc_sh.at[pl.ds(row0 + j * ZROWS, ZROWS)])
            return carry

        lax.fori_loop(0, ROWS_PT // ZROWS, zcopy, 0)
        plsc.subcore_barrier()

        ebase = wid * EPT

        def ebody(ci, carry):
            off = ebase + ci * CH
            pltpu.sync_copy(src_hbm.at[pl.ds(off, CH)], sidx_v)
            pltpu.sync_copy(dst_hbm.at[pl.ds(off, CH)], didx_v)
            pltpu.async_copy(feat_hbm.at[sidx_v], rows_v, sem).wait()
            pltpu.sync_copy(rows_v, acc_sh.at[didx_v], add=True)
            return carry

        lax.fori_loop(0, NCHUNK, ebody, 0)
        plsc.subcore_barrier()

        def wcopy(j, carry):
            r = row0 + j * ZROWS
            pltpu.sync_copy(acc_sh.at[pl.ds(r, ZROWS)], stage_v)
            pltpu.sync_copy(stage_v, out_hbm.at[cid, pl.ds(r, ZROWS)])
            return carry

        lax.fori_loop(0, ROWS_PT // ZROWS, wcopy, 0)

    return k(feature, src, dst)


def _tc_linear(partials, wt, bias):
    RB = 2048

    def mm(p_ref, w_ref, b_ref, o_ref):
        acc = p_ref[0] + p_ref[1]
        o_ref[...] = (
            jnp.dot(acc, w_ref[...], preferred_element_type=jnp.float32)
            + b_ref[...]
        )

    return pl.pallas_call(
        mm,
        grid=(N_PAD // RB,),
        in_specs=[
            pl.BlockSpec((NC, RB, D), lambda i: (0, i, 0)),
            pl.BlockSpec((D, D), lambda i: (0, 0)),
            pl.BlockSpec((1, D), lambda i: (0, 0)),
        ],
        out_specs=pl.BlockSpec((RB, D), lambda i: (i, 0)),
        out_shape=jax.ShapeDtypeStruct((N_PAD, D), jnp.float32),
    )(partials, wt, bias.reshape(1, D))


def kernel(feature, edge_index, W, b):
    ei = edge_index.astype(jnp.int32)
    partials = _sc_segment_sum(feature, ei[0], ei[1])
    return _tc_linear(partials, W.T, b)[:N_NODES]
